# Initial kernel scaffold; baseline (speedup 1.0000x reference)
import jax
import jax.numpy as jnp
from jax.experimental import pallas as pl


def kernel(x, adj, W0, a_src0, a_dst0, W1, a_src1, a_dst1):
    raise NotImplementedError("write your pallas kernel here")


# speedup vs baseline: 2.3748x; 2.3748x over previous
import functools

import jax
import jax.numpy as jnp
from jax.experimental import pallas as pl
from jax.experimental.pallas import tpu as pltpu

N = 4096
D = 128
H = 4
DH = D // H
BI = 256


def _gat_kernel(x_ref, adj_ref, w_ref, asrc_ref, adst_ref, out_ref,
                h_s, fsrc_s, fdst_s):
    i = pl.program_id(0)

    @pl.when(i == 0)
    def _prologue():
        h = jax.lax.dot_general(
            x_ref[...], w_ref[...], (((1,), (0,)), ((), ())),
            preferred_element_type=jnp.float32)
        h_s[...] = h
        fsrc_s[...] = jax.lax.dot_general(
            h, asrc_ref[...], (((1,), (1,)), ((), ())),
            preferred_element_type=jnp.float32)
        fdst_s[...] = jax.lax.dot_general(
            adst_ref[...], h, (((1,), (1,)), ((), ())),
            preferred_element_type=jnp.float32)

    mask = adj_ref[...] > 0.0
    for hh in range(H):
        fs = fsrc_s[pl.ds(i * BI, BI), hh:hh + 1]
        fd = fdst_s[hh:hh + 1, :]
        e = fs + fd
        e = jnp.where(e > 0, e, 0.2 * e)
        e = jnp.where(mask, e, jnp.float32(-9e15))
        m = jnp.max(e, axis=1, keepdims=True)
        p = jnp.exp(e - m)
        s = jnp.sum(p, axis=1, keepdims=True)
        o = jax.lax.dot_general(
            p, h_s[:, hh * DH:(hh + 1) * DH], (((1,), (0,)), ((), ())),
            preferred_element_type=jnp.float32) / s
        out_ref[:, hh * DH:(hh + 1) * DH] = jnp.where(o > 0, o, jnp.exp(o) - 1.0)


@functools.partial(jax.jit, static_argnames=())
def _run(x, adj, W, a_src, a_dst):
    cols = jnp.arange(D)
    head_of_col = cols // DH
    a_src_flat = a_src.reshape(D)
    a_dst_flat = a_dst.reshape(D)
    rows = jnp.arange(8)[:, None]
    sel = rows == head_of_col[None, :]
    A_src = jnp.where(sel, a_src_flat[None, :], 0.0).astype(jnp.float32)
    A_dst = jnp.where(sel, a_dst_flat[None, :], 0.0).astype(jnp.float32)

    grid = (N // BI,)
    return pl.pallas_call(
        _gat_kernel,
        grid=grid,
        in_specs=[
            pl.BlockSpec((N, D), lambda i: (0, 0)),
            pl.BlockSpec((BI, N), lambda i: (i, 0)),
            pl.BlockSpec((D, D), lambda i: (0, 0)),
            pl.BlockSpec((8, D), lambda i: (0, 0)),
            pl.BlockSpec((8, D), lambda i: (0, 0)),
        ],
        out_specs=pl.BlockSpec((BI, D), lambda i: (i, 0)),
        out_shape=jax.ShapeDtypeStruct((N, D), jnp.float32),
        scratch_shapes=[
            pltpu.VMEM((N, D), jnp.float32),
            pltpu.VMEM((N, 8), jnp.float32),
            pltpu.VMEM((8, N), jnp.float32),
        ],
    )(x, adj, W, A_src, A_dst)


def kernel(x, adj, W0, a_src0, a_dst0, W1, a_src1, a_dst1):
    return _run(x, adj, W1, a_src1, a_dst1)


# speedup vs baseline: 4.2089x; 1.7723x over previous
import functools

import jax
import jax.numpy as jnp
from jax.experimental import pallas as pl
from jax.experimental.pallas import tpu as pltpu

N = 4096
D = 128
H = 4
DH = D // H
BI = 256


def _gat_kernel(x_ref, adj_ref, w_ref, asrc_ref, adst_ref, out_ref,
                haug_s, fsrc_s, fdst_s, hsum_s):
    i = pl.program_id(0)

    @pl.when(i == 0)
    def _prologue():
        h = jax.lax.dot_general(
            x_ref[...], w_ref[...], (((1,), (0,)), ((), ())),
            preferred_element_type=jnp.float32)
        for hh in range(H):
            haug_s[:, hh * 2 * DH:hh * 2 * DH + DH] = h[:, hh * DH:(hh + 1) * DH]
            haug_s[:, hh * 2 * DH + DH:(hh + 1) * 2 * DH] = jnp.ones(
                (N, DH), jnp.float32)
        fsrc_s[...] = jax.lax.dot_general(
            h, asrc_ref[...], (((1,), (1,)), ((), ())),
            preferred_element_type=jnp.float32)
        fdst_s[...] = jax.lax.dot_general(
            adst_ref[...], h, (((1,), (1,)), ((), ())),
            preferred_element_type=jnp.float32)
        hsum_s[0:1, :] = jnp.sum(h, axis=0, keepdims=True)

    adj = adj_ref[...]
    for hh in range(H):
        fs = fsrc_s[pl.ds(i * BI, BI), hh:hh + 1]
        fd = fdst_s[hh:hh + 1, :]
        e = fs + fd
        p = jnp.exp(jnp.maximum(e, 0.2 * e)) * adj
        ps = jax.lax.dot_general(
            p, haug_s[:, hh * 2 * DH:(hh + 1) * 2 * DH], (((1,), (0,)), ((), ())),
            preferred_element_type=jnp.float32)
        s = ps[:, DH:DH + 1]
        o = ps[:, :DH] / jnp.maximum(s, jnp.float32(1e-30))
        o = jnp.where(s > 0, o, hsum_s[0:1, hh * DH:(hh + 1) * DH] * (1.0 / N))
        out_ref[:, hh * DH:(hh + 1) * DH] = jnp.where(o > 0, o, jnp.exp(o) - 1.0)


@functools.partial(jax.jit, static_argnames=())
def _run(x, adj, W, a_src, a_dst):
    cols = jnp.arange(D)
    head_of_col = cols // DH
    rows = jnp.arange(8)[:, None]
    sel = rows == head_of_col[None, :]
    A_src = jnp.where(sel, a_src.reshape(D)[None, :], 0.0).astype(jnp.float32)
    A_dst = jnp.where(sel, a_dst.reshape(D)[None, :], 0.0).astype(jnp.float32)

    grid = (N // BI,)
    return pl.pallas_call(
        _gat_kernel,
        grid=grid,
        in_specs=[
            pl.BlockSpec((N, D), lambda i: (0, 0)),
            pl.BlockSpec((BI, N), lambda i: (i, 0)),
            pl.BlockSpec((D, D), lambda i: (0, 0)),
            pl.BlockSpec((8, D), lambda i: (0, 0)),
            pl.BlockSpec((8, D), lambda i: (0, 0)),
        ],
        out_specs=pl.BlockSpec((BI, D), lambda i: (i, 0)),
        out_shape=jax.ShapeDtypeStruct((N, D), jnp.float32),
        scratch_shapes=[
            pltpu.VMEM((N, 2 * D), jnp.float32),
            pltpu.VMEM((N, 8), jnp.float32),
            pltpu.VMEM((8, N), jnp.float32),
            pltpu.VMEM((8, D), jnp.float32),
        ],
    )(x, adj, W, A_src, A_dst)


def kernel(x, adj, W0, a_src0, a_dst0, W1, a_src1, a_dst1):
    return _run(x, adj, W1, a_src1, a_dst1)


# speedup vs baseline: 4.9581x; 1.1780x over previous
import functools

import jax
import jax.numpy as jnp
from jax.experimental import pallas as pl
from jax.experimental.pallas import tpu as pltpu

N = 4096
D = 128
H = 4
DH = D // H
BI = 256


def _gat_kernel(x_ref, adj_ref, w_ref, asrc_ref, adst_ref, out_ref,
                haug_s, esrc_s, edst_s, hsum_s):
    i = pl.program_id(0)

    @pl.when(i == 0)
    def _prologue():
        h = jax.lax.dot_general(
            x_ref[...], w_ref[...], (((1,), (0,)), ((), ())),
            preferred_element_type=jnp.float32)
        for hh in range(H):
            haug_s[:, hh * 2 * DH:hh * 2 * DH + DH] = h[:, hh * DH:(hh + 1) * DH]
            haug_s[:, hh * 2 * DH + DH:(hh + 1) * 2 * DH] = jnp.ones(
                (N, DH), jnp.float32)
        fsrc = jax.lax.dot_general(
            h, asrc_ref[...], (((1,), (1,)), ((), ())),
            preferred_element_type=jnp.float32)
        fdst = jax.lax.dot_general(
            adst_ref[...], h, (((1,), (1,)), ((), ())),
            preferred_element_type=jnp.float32)
        esrc_s[:, 0:8] = jnp.exp(fsrc)
        esrc_s[:, 8:16] = jnp.exp(0.2 * fsrc)
        edst_s[0:8, :] = jnp.exp(fdst)
        edst_s[8:16, :] = jnp.exp(0.2 * fdst)
        hsum_s[0:1, :] = jnp.sum(h, axis=0, keepdims=True)

    adj = adj_ref[...]
    for hh in range(H):
        es1 = esrc_s[pl.ds(i * BI, BI), hh:hh + 1]
        es2 = esrc_s[pl.ds(i * BI, BI), 8 + hh:9 + hh]
        ed1 = edst_s[hh:hh + 1, :]
        ed2 = edst_s[8 + hh:9 + hh, :]
        p = jnp.maximum(es1 * ed1, es2 * ed2) * adj
        ps = jax.lax.dot_general(
            p, haug_s[:, hh * 2 * DH:(hh + 1) * 2 * DH], (((1,), (0,)), ((), ())),
            preferred_element_type=jnp.float32)
        s = ps[:, DH:DH + 1]
        o = ps[:, :DH] / jnp.maximum(s, jnp.float32(1e-30))
        o = jnp.where(s > 0, o, hsum_s[0:1, hh * DH:(hh + 1) * DH] * (1.0 / N))
        out_ref[:, hh * DH:(hh + 1) * DH] = jnp.where(o > 0, o, jnp.exp(o) - 1.0)


@functools.partial(jax.jit, static_argnames=())
def _run(x, adj, W, a_src, a_dst):
    cols = jnp.arange(D)
    head_of_col = cols // DH
    rows = jnp.arange(8)[:, None]
    sel = rows == head_of_col[None, :]
    A_src = jnp.where(sel, a_src.reshape(D)[None, :], 0.0).astype(jnp.float32)
    A_dst = jnp.where(sel, a_dst.reshape(D)[None, :], 0.0).astype(jnp.float32)

    grid = (N // BI,)
    return pl.pallas_call(
        _gat_kernel,
        grid=grid,
        in_specs=[
            pl.BlockSpec((N, D), lambda i: (0, 0)),
            pl.BlockSpec((BI, N), lambda i: (i, 0)),
            pl.BlockSpec((D, D), lambda i: (0, 0)),
            pl.BlockSpec((8, D), lambda i: (0, 0)),
            pl.BlockSpec((8, D), lambda i: (0, 0)),
        ],
        out_specs=pl.BlockSpec((BI, D), lambda i: (i, 0)),
        out_shape=jax.ShapeDtypeStruct((N, D), jnp.float32),
        scratch_shapes=[
            pltpu.VMEM((N, 2 * D), jnp.float32),
            pltpu.VMEM((N, 16), jnp.float32),
            pltpu.VMEM((16, N), jnp.float32),
            pltpu.VMEM((8, D), jnp.float32),
        ],
    )(x, adj, W, A_src, A_dst)


def kernel(x, adj, W0, a_src0, a_dst0, W1, a_src1, a_dst1):
    return _run(x, adj, W1, a_src1, a_dst1)


# speedup vs baseline: 5.3386x; 1.0767x over previous
import functools

import jax
import jax.numpy as jnp
from jax.experimental import pallas as pl
from jax.experimental.pallas import tpu as pltpu

N = 4096
D = 128
H = 4
DH = D // H
BI = 512


def _gat_kernel(x_ref, adj_ref, w_ref, asrc_ref, adst_ref, out_ref,
                haug_s, esrc_s, edst_s, hsum_s):
    i = pl.program_id(0)

    @pl.when(i == 0)
    def _prologue():
        h = jax.lax.dot_general(
            x_ref[...], w_ref[...], (((1,), (0,)), ((), ())),
            preferred_element_type=jnp.float32)
        for hh in range(H):
            haug_s[:, hh * 2 * DH:hh * 2 * DH + DH] = h[:, hh * DH:(hh + 1) * DH]
            haug_s[:, hh * 2 * DH + DH:(hh + 1) * 2 * DH] = jnp.ones(
                (N, DH), jnp.float32)
        fsrc = jax.lax.dot_general(
            h, asrc_ref[...], (((1,), (1,)), ((), ())),
            preferred_element_type=jnp.float32)
        fdst = jax.lax.dot_general(
            adst_ref[...], h, (((1,), (1,)), ((), ())),
            preferred_element_type=jnp.float32)
        esrc_s[:, 0:8] = jnp.exp(fsrc)
        esrc_s[:, 8:16] = jnp.exp(0.2 * fsrc)
        edst_s[0:8, :] = jnp.exp(fdst)
        edst_s[8:16, :] = jnp.exp(0.2 * fdst)
        hsum_s[0:1, :] = jnp.sum(h, axis=0, keepdims=True)

    adj = adj_ref[...]
    for hh in range(H):
        es1 = esrc_s[pl.ds(i * BI, BI), hh:hh + 1]
        es2 = esrc_s[pl.ds(i * BI, BI), 8 + hh:9 + hh]
        ed1 = edst_s[hh:hh + 1, :]
        ed2 = edst_s[8 + hh:9 + hh, :]
        p = jnp.maximum(es1 * ed1, es2 * ed2) * adj
        ps = jax.lax.dot_general(
            p, haug_s[:, hh * 2 * DH:(hh + 1) * 2 * DH], (((1,), (0,)), ((), ())),
            preferred_element_type=jnp.float32)
        s = ps[:, DH:DH + 1]
        o = ps[:, :DH] / jnp.maximum(s, jnp.float32(1e-30))
        o = jnp.where(s > 0, o, hsum_s[0:1, hh * DH:(hh + 1) * DH] * (1.0 / N))
        out_ref[:, hh * DH:(hh + 1) * DH] = jnp.where(o > 0, o, jnp.exp(o) - 1.0)


@functools.partial(jax.jit, static_argnames=())
def _run(x, adj, W, a_src, a_dst):
    cols = jnp.arange(D)
    head_of_col = cols // DH
    rows = jnp.arange(8)[:, None]
    sel = rows == head_of_col[None, :]
    A_src = jnp.where(sel, a_src.reshape(D)[None, :], 0.0).astype(jnp.float32)
    A_dst = jnp.where(sel, a_dst.reshape(D)[None, :], 0.0).astype(jnp.float32)

    grid = (N // BI,)
    return pl.pallas_call(
        _gat_kernel,
        grid=grid,
        in_specs=[
            pl.BlockSpec((N, D), lambda i: (0, 0)),
            pl.BlockSpec((BI, N), lambda i: (i, 0)),
            pl.BlockSpec((D, D), lambda i: (0, 0)),
            pl.BlockSpec((8, D), lambda i: (0, 0)),
            pl.BlockSpec((8, D), lambda i: (0, 0)),
        ],
        out_specs=pl.BlockSpec((BI, D), lambda i: (i, 0)),
        out_shape=jax.ShapeDtypeStruct((N, D), jnp.float32),
        scratch_shapes=[
            pltpu.VMEM((N, 2 * D), jnp.float32),
            pltpu.VMEM((N, 16), jnp.float32),
            pltpu.VMEM((16, N), jnp.float32),
            pltpu.VMEM((8, D), jnp.float32),
        ],
    )(x, adj, W, A_src, A_dst)


def kernel(x, adj, W0, a_src0, a_dst0, W1, a_src1, a_dst1):
    return _run(x, adj, W1, a_src1, a_dst1)


# speedup vs baseline: 6.1339x; 1.1490x over previous
import functools

import jax
import jax.numpy as jnp
from jax.experimental import pallas as pl
from jax.experimental.pallas import tpu as pltpu

N = 4096
D = 128
H = 4
DH = D // H
BI = 512


def _gat_kernel(x_ref, adj_ref, w_ref, asrc_ref, adst_ref, out_ref,
                haug_s, esrc_s, edst_s, hsum_s):
    i = pl.program_id(0)

    @pl.when(i == 0)
    def _prologue():
        h = jax.lax.dot_general(
            x_ref[...], w_ref[...], (((1,), (0,)), ((), ())),
            preferred_element_type=jnp.float32)
        for hh in range(H):
            haug_s[:, hh * 2 * DH:hh * 2 * DH + DH] = h[:, hh * DH:(hh + 1) * DH]
            haug_s[:, hh * 2 * DH + DH:(hh + 1) * 2 * DH] = jnp.ones(
                (N, DH), jnp.float32)
        fsrc = jax.lax.dot_general(
            h, asrc_ref[...], (((1,), (1,)), ((), ())),
            preferred_element_type=jnp.float32)
        fdst = jax.lax.dot_general(
            adst_ref[...], h, (((1,), (1,)), ((), ())),
            preferred_element_type=jnp.float32)
        esrc_s[:, 0:8] = jnp.exp(-0.8 * fsrc)
        edst_s[0:8, :] = jnp.exp(fdst)
        edst_s[8:16, :] = jnp.exp(0.2 * fdst)
        hsum_s[0:1, :] = jnp.sum(h, axis=0, keepdims=True)

    adj = adj_ref[...]
    for hh in range(H):
        r = esrc_s[pl.ds(i * BI, BI), hh:hh + 1]
        ed1 = edst_s[hh:hh + 1, :]
        ed2 = edst_s[8 + hh:9 + hh, :]
        p = jnp.maximum(ed1, r * ed2) * adj
        ps = jax.lax.dot_general(
            p, haug_s[:, hh * 2 * DH:(hh + 1) * 2 * DH], (((1,), (0,)), ((), ())),
            preferred_element_type=jnp.float32)
        s = ps[:, DH:DH + 1]
        o = ps[:, :DH] / jnp.maximum(s, jnp.float32(1e-30))
        o = jnp.where(s > 0, o, hsum_s[0:1, hh * DH:(hh + 1) * DH] * (1.0 / N))
        out_ref[:, hh * DH:(hh + 1) * DH] = jnp.where(o > 0, o, jnp.exp(o) - 1.0)


@functools.partial(jax.jit, static_argnames=())
def _run(x, adj, W, a_src, a_dst):
    cols = jnp.arange(D)
    head_of_col = cols // DH
    rows = jnp.arange(8)[:, None]
    sel = rows == head_of_col[None, :]
    A_src = jnp.where(sel, a_src.reshape(D)[None, :], 0.0).astype(jnp.float32)
    A_dst = jnp.where(sel, a_dst.reshape(D)[None, :], 0.0).astype(jnp.float32)

    grid = (N // BI,)
    return pl.pallas_call(
        _gat_kernel,
        grid=grid,
        in_specs=[
            pl.BlockSpec((N, D), lambda i: (0, 0)),
            pl.BlockSpec((BI, N), lambda i: (i, 0)),
            pl.BlockSpec((D, D), lambda i: (0, 0)),
            pl.BlockSpec((8, D), lambda i: (0, 0)),
            pl.BlockSpec((8, D), lambda i: (0, 0)),
        ],
        out_specs=pl.BlockSpec((BI, D), lambda i: (i, 0)),
        out_shape=jax.ShapeDtypeStruct((N, D), jnp.float32),
        scratch_shapes=[
            pltpu.VMEM((N, 2 * D), jnp.float32),
            pltpu.VMEM((N, 8), jnp.float32),
            pltpu.VMEM((16, N), jnp.float32),
            pltpu.VMEM((8, D), jnp.float32),
        ],
    )(x, adj, W, A_src, A_dst)


def kernel(x, adj, W0, a_src0, a_dst0, W1, a_src1, a_dst1):
    return _run(x, adj, W1, a_src1, a_dst1)
